# baseline (speedup 1.0000x reference)
import functools

import jax
import jax.numpy as jnp
from jax import lax
from jax.experimental import pallas as pl
from jax.experimental.pallas import tpu as pltpu
from jax.experimental.pallas import tpu_sc as plsc

V, D = 30522, 768
B, L = 4096, 200
NOUT = 10
DP = 16


_BM = 1024


VP = 30528
_PACK = 128 // DP


def _tw_body(t_ref, w_ref, o_ref):
    p = jnp.dot(t_ref[...], w_ref[...], preferred_element_type=jnp.float32)
    o_ref[...] = p.T


def _table_times_w(table, w_pad):
    return pl.pallas_call(
        _tw_body,
        grid=(pl.cdiv(V, _BM),),
        in_specs=[
            pl.BlockSpec((_BM, D), lambda i: (i, 0)),
            pl.BlockSpec((D, DP), lambda i: (0, 0)),
        ],
        out_specs=pl.BlockSpec((DP, _BM), lambda i: (0, i)),
        out_shape=jax.ShapeDtypeStruct((DP, VP), jnp.float32),
    )(table, w_pad)


NC, NS = 2, 16
NW = NC * NS
BPW = B // NW
CB = 8
CI = CB * L
NCHUNK = BPW // CB
DESC = 100
NDESC = CI // DESC
INV_L = 1.0 / L
_UNROLL = 8


@functools.lru_cache(maxsize=1)
def _make_sc_pool():
    mesh = plsc.VectorSubcoreMesh(core_axis_name="c", subcore_axis_name="s")

    @functools.partial(
        pl.kernel,
        mesh=mesh,
        out_type=jax.ShapeDtypeStruct((B, DP), jnp.float32),
        compiler_params=pltpu.CompilerParams(use_tc_tiling_on_sc=False),
        scratch_types=[
            pltpu.VMEM((NCHUNK * NDESC, DESC), jnp.int32),
            pltpu.VMEM((CI, DP), jnp.float32),
            pltpu.VMEM((CI, DP), jnp.float32),
            pltpu.VMEM((BPW, DP), jnp.float32),
            pltpu.VMEM((DP,), jnp.float32),
            pltpu.SemaphoreType.DMA,
            pltpu.SemaphoreType.DMA,
        ],
    )
    def k(x_hbm, tw_hbm, b_hbm, out_hbm, idx_v, buf0, buf1, out_v, b_v,
          sem0, sem1):
        wid = lax.axis_index("s") * NC + lax.axis_index("c")
        pltpu.sync_copy(x_hbm.at[wid], idx_v)
        pltpu.sync_copy(b_hbm, b_v)

        def chunk_copies(c, buf, sem):
            cps = []
            for dnum in range(NDESC):
                src = tw_hbm.at[idx_v.at[c * NDESC + dnum]]
                dst = buf.at[pl.ds(dnum * DESC, DESC)]
                cps.append(pltpu.make_async_copy(src, dst, sem))
            return cps

        def start_chunk(c, buf, sem):
            for cp in chunk_copies(c, buf, sem):
                cp.start()

        def wait_chunk(c, buf, sem):
            for cp in chunk_copies(c, buf, sem):
                cp.wait()

        def accum_chunk(c, buf):
            for r in range(CB):
                def jbody(j, accs, _r=r):
                    o = _r * L + j * _UNROLL
                    return tuple(accs[u] + buf[o + u] for u in range(_UNROLL))
                accs = lax.fori_loop(
                    0, L // _UNROLL, jbody,
                    tuple(jnp.zeros((DP,), jnp.float32)
                          for _ in range(_UNROLL)))
                s = (((accs[0] + accs[1]) + (accs[2] + accs[3]))
                     + ((accs[4] + accs[5]) + (accs[6] + accs[7])))
                out_v[c * CB + r] = s * INV_L + b_v[...]

        start_chunk(0, buf0, sem0)

        def body(cc, carry):
            ca = 2 * cc
            start_chunk(ca + 1, buf1, sem1)
            wait_chunk(ca, buf0, sem0)
            accum_chunk(ca, buf0)

            @pl.when(cc < NCHUNK // 2 - 1)
            def _():
                start_chunk(ca + 2, buf0, sem0)

            wait_chunk(ca + 1, buf1, sem1)
            accum_chunk(ca + 1, buf1)
            return carry

        lax.fori_loop(0, NCHUNK // 2, body, 0)
        pltpu.sync_copy(out_v, out_hbm.at[pl.ds(wid * BPW, BPW)])

    return k


def kernel(x, table, W, b):
    w_pad = jnp.pad(W, ((0, 0), (0, DP - NOUT)))
    b_pad = jnp.pad(b, (0, DP - NOUT))
    tw = _table_times_w(table, w_pad).T
    xr = x.reshape(NW, NCHUNK * NDESC, DESC)
    out_pad = _make_sc_pool()(xr, tw, b_pad)
    return out_pad[:, :NOUT]


# speedup vs baseline: 1.0749x; 1.0749x over previous
import functools

import jax
import jax.numpy as jnp
from jax import lax
from jax.experimental import pallas as pl
from jax.experimental.pallas import tpu as pltpu
from jax.experimental.pallas import tpu_sc as plsc

V, D = 30522, 768
B, L = 4096, 200
NOUT = 10
DP = 16


_BM = 1024


def _tw_body(t_ref, w_ref, o_ref):
    w16 = jnp.concatenate(
        [w_ref[...], jnp.zeros((D, DP - NOUT), jnp.float32)], axis=1)
    o_ref[...] = jnp.dot(t_ref[...], w16, preferred_element_type=jnp.float32)


def _table_times_w(table, w):
    return pl.pallas_call(
        _tw_body,
        grid=(pl.cdiv(V, _BM),),
        in_specs=[
            pl.BlockSpec((_BM, D), lambda i: (i, 0)),
            pl.BlockSpec((D, NOUT), lambda i: (0, 0)),
        ],
        out_specs=pl.BlockSpec((_BM, DP), lambda i: (i, 0)),
        out_shape=jax.ShapeDtypeStruct((V, DP), jnp.float32),
    )(table, w)


NC, NS = 2, 16
NW = NC * NS
BPW = B // NW
CB = 16
CI = CB * L
NCHUNK = BPW // CB
DESC = 128
CDESC = CI // DESC
NDESC_W = BPW * L // DESC
INV_L = 1.0 / L
_UNROLL = 8


@functools.lru_cache(maxsize=1)
def _make_sc_pool():
    mesh = plsc.VectorSubcoreMesh(core_axis_name="c", subcore_axis_name="s")

    @functools.partial(
        pl.kernel,
        mesh=mesh,
        out_type=jax.ShapeDtypeStruct((B, DP), jnp.float32),
        compiler_params=pltpu.CompilerParams(use_tc_tiling_on_sc=False),
        scratch_types=[
            pltpu.VMEM((NDESC_W, DESC), jnp.int32),
            pltpu.VMEM((CI, DP), jnp.float32),
            pltpu.VMEM((CI, DP), jnp.float32),
            pltpu.VMEM((BPW, DP), jnp.float32),
            pltpu.VMEM((DP,), jnp.float32),
            pltpu.SemaphoreType.DMA,
            pltpu.SemaphoreType.DMA,
        ],
    )
    def k(x_hbm, tw_hbm, b_hbm, out_hbm, idx_v, buf0, buf1, out_v, b_v,
          sem0, sem1):
        wid = lax.axis_index("s") * NC + lax.axis_index("c")
        pltpu.sync_copy(x_hbm.at[wid], idx_v)
        pltpu.sync_copy(b_hbm, b_v.at[pl.ds(0, NOUT)])

        def chunk_copies(c, buf, sem):
            cps = []
            for dnum in range(CDESC):
                src = tw_hbm.at[idx_v.at[c * CDESC + dnum]]
                dst = buf.at[pl.ds(dnum * DESC, DESC)]
                cps.append(pltpu.make_async_copy(src, dst, sem))
            return cps

        def start_chunk(c, buf, sem):
            for cp in chunk_copies(c, buf, sem):
                cp.start()

        def wait_chunk(c, buf, sem):
            for cp in chunk_copies(c, buf, sem):
                cp.wait()

        def accum_chunk(c, buf):
            for r in range(CB):
                def jbody(j, accs, _r=r):
                    o = _r * L + j * _UNROLL
                    return tuple(accs[u] + buf[o + u] for u in range(_UNROLL))
                accs = lax.fori_loop(
                    0, L // _UNROLL, jbody,
                    tuple(jnp.zeros((DP,), jnp.float32)
                          for _ in range(_UNROLL)))
                s = (((accs[0] + accs[1]) + (accs[2] + accs[3]))
                     + ((accs[4] + accs[5]) + (accs[6] + accs[7])))
                out_v[c * CB + r] = s * INV_L + b_v[...]

        start_chunk(0, buf0, sem0)

        def body(cc, carry):
            ca = 2 * cc
            start_chunk(ca + 1, buf1, sem1)
            wait_chunk(ca, buf0, sem0)
            accum_chunk(ca, buf0)

            @pl.when(cc < NCHUNK // 2 - 1)
            def _():
                start_chunk(ca + 2, buf0, sem0)

            wait_chunk(ca + 1, buf1, sem1)
            accum_chunk(ca + 1, buf1)
            return carry

        lax.fori_loop(0, NCHUNK // 2, body, 0)
        pltpu.sync_copy(out_v, out_hbm.at[pl.ds(wid * BPW, BPW)])

    return k


def kernel(x, table, W, b):
    tw = _table_times_w(table, W)
    xr = x.reshape(NW, NDESC_W, DESC)
    out_pad = _make_sc_pool()(xr, tw, b)
    return out_pad[:, :NOUT]


# speedup vs baseline: 1.1941x; 1.1109x over previous
import functools

import jax
import jax.numpy as jnp
from jax import lax
from jax.experimental import pallas as pl
from jax.experimental.pallas import tpu as pltpu
from jax.experimental.pallas import tpu_sc as plsc

V, D = 30522, 768
B, L = 4096, 200
NOUT = 10
DP = 16


_BM = 1024


VP = 30528
_PACK = 128 // DP


def _tw_body(t_ref, w_ref, o_ref):
    w16 = jnp.concatenate(
        [w_ref[...], jnp.zeros((D, DP - NOUT), jnp.float32)], axis=1)
    p = jnp.dot(t_ref[...], w16, preferred_element_type=jnp.float32)
    p3 = p.reshape(_BM // _PACK, _PACK, DP)
    o_ref[...] = jnp.concatenate([p3[:, r, :] for r in range(_PACK)], axis=1)


def _table_times_w(table, w):
    return pl.pallas_call(
        _tw_body,
        grid=(pl.cdiv(V, _BM),),
        in_specs=[
            pl.BlockSpec((_BM, D), lambda i: (i, 0)),
            pl.BlockSpec((D, NOUT), lambda i: (0, 0)),
        ],
        out_specs=pl.BlockSpec((_BM // _PACK, 128), lambda i: (i, 0)),
        out_shape=jax.ShapeDtypeStruct((VP // _PACK, 128), jnp.float32),
    )(table, w)


NC, NS = 2, 16
NW = NC * NS
BPW = B // NW
CB = 16
CI = CB * L
NCHUNK = BPW // CB
DESC = 128
CDESC = CI // DESC
NDESC_W = BPW * L // DESC
INV_L = 1.0 / L
_UNROLL = 8


@functools.lru_cache(maxsize=1)
def _make_sc_pool():
    mesh = plsc.VectorSubcoreMesh(core_axis_name="c", subcore_axis_name="s")

    @functools.partial(
        pl.kernel,
        mesh=mesh,
        out_type=jax.ShapeDtypeStruct((B, DP), jnp.float32),
        compiler_params=pltpu.CompilerParams(use_tc_tiling_on_sc=False),
        scratch_types=[
            pltpu.VMEM((NDESC_W, DESC), jnp.int32),
            pltpu.VMEM((CI, DP), jnp.float32),
            pltpu.VMEM((CI, DP), jnp.float32),
            pltpu.VMEM((BPW, DP), jnp.float32),
            pltpu.VMEM((DP,), jnp.float32),
            pltpu.SemaphoreType.DMA,
            pltpu.SemaphoreType.DMA,
        ],
    )
    def k(x_hbm, tw_hbm, b_hbm, out_hbm, idx_v, buf0, buf1, out_v, b_v,
          sem0, sem1):
        wid = lax.axis_index("s") * NC + lax.axis_index("c")
        pltpu.sync_copy(x_hbm.at[wid], idx_v)
        pltpu.sync_copy(b_hbm, b_v.at[pl.ds(0, NOUT)])

        def chunk_copies(c, buf, sem):
            cps = []
            for dnum in range(CDESC):
                src = tw_hbm.at[idx_v.at[c * CDESC + dnum]]
                dst = buf.at[pl.ds(dnum * DESC, DESC)]
                cps.append(pltpu.make_async_copy(src, dst, sem))
            return cps

        def start_chunk(c, buf, sem):
            for cp in chunk_copies(c, buf, sem):
                cp.start()

        def wait_chunk(c, buf, sem):
            for cp in chunk_copies(c, buf, sem):
                cp.wait()

        def accum_chunk(c, buf):
            for r in range(CB):
                def jbody(j, accs, _r=r):
                    o = _r * L + j * _UNROLL
                    return tuple(accs[u] + buf[o + u] for u in range(_UNROLL))
                accs = lax.fori_loop(
                    0, L // _UNROLL, jbody,
                    tuple(jnp.zeros((DP,), jnp.float32)
                          for _ in range(_UNROLL)))
                s = (((accs[0] + accs[1]) + (accs[2] + accs[3]))
                     + ((accs[4] + accs[5]) + (accs[6] + accs[7])))
                out_v[c * CB + r] = s * INV_L + b_v[...]

        start_chunk(0, buf0, sem0)

        def body(cc, carry):
            ca = 2 * cc
            start_chunk(ca + 1, buf1, sem1)
            wait_chunk(ca, buf0, sem0)
            accum_chunk(ca, buf0)

            @pl.when(cc < NCHUNK // 2 - 1)
            def _():
                start_chunk(ca + 2, buf0, sem0)

            wait_chunk(ca + 1, buf1, sem1)
            accum_chunk(ca + 1, buf1)
            return carry

        lax.fori_loop(0, NCHUNK // 2, body, 0)
        pltpu.sync_copy(out_v, out_hbm.at[pl.ds(wid * BPW, BPW)])

    return k


def kernel(x, table, W, b):
    tw = _table_times_w(table, W).reshape(VP, DP)
    xr = x.reshape(NW, NDESC_W, DESC)
    out_pad = _make_sc_pool()(xr, tw, b)
    return out_pad[:, :NOUT]


# speedup vs baseline: 1.3186x; 1.1042x over previous
import functools

import jax
import jax.numpy as jnp
from jax import lax
from jax.experimental import pallas as pl
from jax.experimental.pallas import tpu as pltpu
from jax.experimental.pallas import tpu_sc as plsc

V, D = 30522, 768
B, L = 4096, 200
NOUT = 10
DP = 16


_BM = 2048


VP = 30528
_PACK = 128 // DP


def _tw_body(t_ref, wt_ref, o_ref):
    p10 = lax.dot_general(t_ref[...], wt_ref[...], (((1,), (1,)), ((), ())),
                          preferred_element_type=jnp.float32)
    p = jnp.concatenate(
        [p10, jnp.zeros((_BM, DP - NOUT), jnp.float32)], axis=1)
    p3 = p.reshape(_BM // _PACK, _PACK, DP)
    o_ref[...] = jnp.concatenate([p3[:, r, :] for r in range(_PACK)], axis=1)


def _table_times_w(table, wt):
    return pl.pallas_call(
        _tw_body,
        grid=(pl.cdiv(V, _BM),),
        in_specs=[
            pl.BlockSpec((_BM, D), lambda i: (i, 0)),
            pl.BlockSpec((NOUT, D), lambda i: (0, 0)),
        ],
        out_specs=pl.BlockSpec((_BM // _PACK, 128), lambda i: (i, 0)),
        out_shape=jax.ShapeDtypeStruct((VP // _PACK, 128), jnp.float32),
    )(table, wt)


NC, NS = 2, 16
NW = NC * NS
BPW = B // NW
CB = 16
CI = CB * L
NCHUNK = BPW // CB
DESC = 128
CDESC = CI // DESC
NDESC_W = BPW * L // DESC
INV_L = 1.0 / L
_UNROLL = 8


@functools.lru_cache(maxsize=1)
def _make_sc_pool():
    mesh = plsc.VectorSubcoreMesh(core_axis_name="c", subcore_axis_name="s")

    @functools.partial(
        pl.kernel,
        mesh=mesh,
        out_type=jax.ShapeDtypeStruct((B, DP), jnp.float32),
        compiler_params=pltpu.CompilerParams(use_tc_tiling_on_sc=False),
        scratch_types=[
            pltpu.VMEM((NDESC_W, DESC), jnp.int32),
            pltpu.VMEM((CI, DP), jnp.float32),
            pltpu.VMEM((CI, DP), jnp.float32),
            pltpu.VMEM((BPW, DP), jnp.float32),
            pltpu.VMEM((DP,), jnp.float32),
            pltpu.SemaphoreType.DMA,
            pltpu.SemaphoreType.DMA,
        ],
    )
    def k(x_hbm, tw_hbm, b_hbm, out_hbm, idx_v, buf0, buf1, out_v, b_v,
          sem0, sem1):
        wid = lax.axis_index("s") * NC + lax.axis_index("c")
        pltpu.sync_copy(x_hbm.at[wid], idx_v)
        pltpu.sync_copy(b_hbm, b_v.at[pl.ds(0, NOUT)])

        def chunk_copies(c, buf, sem):
            cps = []
            for dnum in range(CDESC):
                src = tw_hbm.at[idx_v.at[c * CDESC + dnum]]
                dst = buf.at[pl.ds(dnum * DESC, DESC)]
                cps.append(pltpu.make_async_copy(src, dst, sem))
            return cps

        def start_chunk(c, buf, sem):
            for cp in chunk_copies(c, buf, sem):
                cp.start()

        def wait_chunk(c, buf, sem):
            for cp in chunk_copies(c, buf, sem):
                cp.wait()

        def accum_chunk(c, buf):
            for r in range(CB):
                def jbody(j, accs, _r=r):
                    o = _r * L + j * _UNROLL
                    return tuple(accs[u] + buf[o + u] for u in range(_UNROLL))
                accs = lax.fori_loop(
                    0, L // _UNROLL, jbody,
                    tuple(jnp.zeros((DP,), jnp.float32)
                          for _ in range(_UNROLL)))
                s = (((accs[0] + accs[1]) + (accs[2] + accs[3]))
                     + ((accs[4] + accs[5]) + (accs[6] + accs[7])))
                out_v[c * CB + r] = s * INV_L + b_v[...]

        start_chunk(0, buf0, sem0)

        def body(cc, carry):
            ca = 2 * cc
            start_chunk(ca + 1, buf1, sem1)
            wait_chunk(ca, buf0, sem0)
            accum_chunk(ca, buf0)

            @pl.when(cc < NCHUNK // 2 - 1)
            def _():
                start_chunk(ca + 2, buf0, sem0)

            wait_chunk(ca + 1, buf1, sem1)
            accum_chunk(ca + 1, buf1)
            return carry

        lax.fori_loop(0, NCHUNK // 2, body, 0)
        pltpu.sync_copy(out_v, out_hbm.at[pl.ds(wid * BPW, BPW)])

    return k


def kernel(x, table, W, b):
    tw = _table_times_w(table, W.T).reshape(VP, DP)
    xr = x.reshape(NW, NDESC_W, DESC)
    out_pad = _make_sc_pool()(xr, tw, b)
    return out_pad[:, :NOUT]


# speedup vs baseline: 1.4141x; 1.0724x over previous
import functools

import jax
import jax.numpy as jnp
from jax import lax
from jax.experimental import pallas as pl
from jax.experimental.pallas import tpu as pltpu
from jax.experimental.pallas import tpu_sc as plsc

V, D = 30522, 768
B, L = 4096, 200
NOUT = 10
DP = 16


_BM = 2048


VP = 30528
_PACK = 128 // DP


def _tw_body(t_ref, wt_ref, o_ref):
    p10 = lax.dot_general(t_ref[...], wt_ref[...], (((1,), (1,)), ((), ())),
                          preferred_element_type=jnp.float32)
    p = jnp.concatenate(
        [p10, jnp.zeros((_BM, DP - NOUT), jnp.float32)], axis=1)
    p3 = p.reshape(_BM // _PACK, _PACK, DP)
    o_ref[...] = jnp.concatenate([p3[:, r, :] for r in range(_PACK)], axis=1)


def _table_times_w(table, wt):
    return pl.pallas_call(
        _tw_body,
        grid=(pl.cdiv(V, _BM),),
        in_specs=[
            pl.BlockSpec((_BM, D), lambda i: (i, 0)),
            pl.BlockSpec((NOUT, D), lambda i: (0, 0)),
        ],
        out_specs=pl.BlockSpec((_BM // _PACK, 128), lambda i: (i, 0)),
        out_shape=jax.ShapeDtypeStruct((VP // _PACK, 128), jnp.float32),
    )(table, wt)


NC, NS = 2, 16
NW = NC * NS
BPW = B // NW
CT = 25
CI = CT * BPW
NCHUNK = L // CT
INV_L = 1.0 / L


@functools.lru_cache(maxsize=1)
def _make_sc_pool():
    mesh = plsc.VectorSubcoreMesh(core_axis_name="c", subcore_axis_name="s")

    @functools.partial(
        pl.kernel,
        mesh=mesh,
        out_type=jax.ShapeDtypeStruct((B, DP), jnp.float32),
        compiler_params=pltpu.CompilerParams(use_tc_tiling_on_sc=False),
        scratch_types=[
            pltpu.VMEM((L, BPW), jnp.int32),
            pltpu.VMEM((CI, DP), jnp.float32),
            pltpu.VMEM((CI, DP), jnp.float32),
            pltpu.VMEM((BPW, DP), jnp.float32),
            pltpu.VMEM((DP,), jnp.float32),
            pltpu.SemaphoreType.DMA,
            pltpu.SemaphoreType.DMA,
        ],
    )
    def k(xt_hbm, tw_hbm, b_hbm, out_hbm, idx_v, buf0, buf1, out_v, b_v,
          sem0, sem1):
        wid = lax.axis_index("s") * NC + lax.axis_index("c")
        base = wid * BPW
        pltpu.sync_copy(xt_hbm.at[:, pl.ds(base, BPW)], idx_v)
        pltpu.sync_copy(b_hbm, b_v.at[pl.ds(0, NOUT)])

        def chunk_copies(c, buf, sem):
            cps = []
            for t in range(CT):
                src = tw_hbm.at[idx_v.at[c * CT + t]]
                dst = buf.at[pl.ds(t * BPW, BPW)]
                cps.append(pltpu.make_async_copy(src, dst, sem))
            return cps

        def start_chunk(c, buf, sem):
            for cp in chunk_copies(c, buf, sem):
                cp.start()

        def wait_chunk(c, buf, sem):
            for cp in chunk_copies(c, buf, sem):
                cp.wait()

        def accum_chunk(buf):
            def rbody(r, carry):
                accs = [buf[t * BPW + r] for t in range(5)]
                for t in range(5, CT):
                    accs[t % 5] = accs[t % 5] + buf[t * BPW + r]
                s = (accs[0] + accs[1]) + (accs[2] + accs[3]) + accs[4]
                out_v[r] = out_v[r] + s
                return carry

            lax.fori_loop(0, BPW, rbody, 0)

        def init_out(r, carry):
            out_v[r] = jnp.zeros((DP,), jnp.float32)
            return carry

        lax.fori_loop(0, BPW, init_out, 0)

        start_chunk(0, buf0, sem0)

        def body(cc, carry):
            ca = 2 * cc
            start_chunk(ca + 1, buf1, sem1)
            wait_chunk(ca, buf0, sem0)
            accum_chunk(buf0)

            @pl.when(cc < NCHUNK // 2 - 1)
            def _():
                start_chunk(ca + 2, buf0, sem0)

            wait_chunk(ca + 1, buf1, sem1)
            accum_chunk(buf1)
            return carry

        lax.fori_loop(0, NCHUNK // 2, body, 0)

        def scale_out(r, carry):
            out_v[r] = out_v[r] * INV_L + b_v[...]
            return carry

        lax.fori_loop(0, BPW, scale_out, 0)
        pltpu.sync_copy(out_v, out_hbm.at[pl.ds(base, BPW)])

    return k


def kernel(x, table, W, b):
    tw = _table_times_w(table, W.T).reshape(VP, DP)
    out_pad = _make_sc_pool()(x.T, tw, b)
    return out_pad[:, :NOUT]


# speedup vs baseline: 1.4307x; 1.0117x over previous
import functools

import jax
import jax.numpy as jnp
from jax import lax
from jax.experimental import pallas as pl
from jax.experimental.pallas import tpu as pltpu
from jax.experimental.pallas import tpu_sc as plsc

V, D = 30522, 768
B, L = 4096, 200
NOUT = 10
DP = 16


_BM = 4096


VP = 30528
_PACK = 128 // DP


def _tw_body(t_ref, wt_ref, o_ref):
    p10 = lax.dot_general(t_ref[...], wt_ref[...] * INV_L,
                          (((1,), (1,)), ((), ())),
                          preferred_element_type=jnp.float32)
    p = jnp.concatenate(
        [p10, jnp.zeros((_BM, DP - NOUT), jnp.float32)], axis=1)
    p3 = p.reshape(_BM // _PACK, _PACK, DP)
    o_ref[...] = jnp.concatenate([p3[:, r, :] for r in range(_PACK)], axis=1)


def _table_times_w(table, wt):
    return pl.pallas_call(
        _tw_body,
        grid=(pl.cdiv(V, _BM),),
        in_specs=[
            pl.BlockSpec((_BM, D), lambda i: (i, 0)),
            pl.BlockSpec((NOUT, D), lambda i: (0, 0)),
        ],
        out_specs=pl.BlockSpec((_BM // _PACK, 128), lambda i: (i, 0)),
        out_shape=jax.ShapeDtypeStruct((VP // _PACK, 128), jnp.float32),
    )(table, wt)


NC, NS = 2, 16
NW = NC * NS
BPW = B // NW
CT = 25
CI = CT * BPW
NCHUNK = L // CT
INV_L = 1.0 / L


@functools.lru_cache(maxsize=1)
def _make_sc_pool():
    mesh = plsc.VectorSubcoreMesh(core_axis_name="c", subcore_axis_name="s")

    @functools.partial(
        pl.kernel,
        mesh=mesh,
        out_type=jax.ShapeDtypeStruct((B, DP), jnp.float32),
        compiler_params=pltpu.CompilerParams(use_tc_tiling_on_sc=False),
        scratch_types=[
            pltpu.VMEM((L, BPW), jnp.int32),
            pltpu.VMEM((CI, DP), jnp.float32),
            pltpu.VMEM((CI, DP), jnp.float32),
            pltpu.VMEM((BPW, DP), jnp.float32),
            pltpu.VMEM((DP,), jnp.float32),
            pltpu.SemaphoreType.DMA,
            pltpu.SemaphoreType.DMA,
        ],
    )
    def k(xt_hbm, tw_hbm, b_hbm, out_hbm, idx_v, buf0, buf1, out_v, b_v,
          sem0, sem1):
        wid = lax.axis_index("s") * NC + lax.axis_index("c")
        base = wid * BPW
        pltpu.sync_copy(xt_hbm.at[:, pl.ds(base, BPW)], idx_v)
        pltpu.sync_copy(b_hbm, b_v.at[pl.ds(0, NOUT)])

        def chunk_copies(c, buf, sem):
            cps = []
            for t in range(CT):
                src = tw_hbm.at[idx_v.at[c * CT + t]]
                dst = buf.at[pl.ds(t * BPW, BPW)]
                cps.append(pltpu.make_async_copy(src, dst, sem))
            return cps

        def start_chunk(c, buf, sem):
            for cp in chunk_copies(c, buf, sem):
                cp.start()

        def wait_chunk(c, buf, sem):
            for cp in chunk_copies(c, buf, sem):
                cp.wait()

        def accum_chunk(buf):
            def rbody(i, carry):
                for q in range(2):
                    r = 2 * i + q
                    accs = [buf[t * BPW + r] for t in range(5)]
                    for t in range(5, CT):
                        accs[t % 5] = accs[t % 5] + buf[t * BPW + r]
                    s = (accs[0] + accs[1]) + (accs[2] + accs[3]) + accs[4]
                    out_v[r] = out_v[r] + s
                return carry

            lax.fori_loop(0, BPW // 2, rbody, 0)

        def init_out(r, carry):
            out_v[r] = b_v[...]
            return carry

        lax.fori_loop(0, BPW, init_out, 0)

        start_chunk(0, buf0, sem0)

        def body(cc, carry):
            ca = 2 * cc
            start_chunk(ca + 1, buf1, sem1)
            wait_chunk(ca, buf0, sem0)
            accum_chunk(buf0)

            @pl.when(cc < NCHUNK // 2 - 1)
            def _():
                start_chunk(ca + 2, buf0, sem0)

            wait_chunk(ca + 1, buf1, sem1)
            accum_chunk(buf1)
            return carry

        lax.fori_loop(0, NCHUNK // 2, body, 0)
        pltpu.sync_copy(out_v, out_hbm.at[pl.ds(base, BPW)])

    return k


def kernel(x, table, W, b):
    tw = _table_times_w(table, W.T).reshape(VP, DP)
    out_pad = _make_sc_pool()(x.T, tw, b)
    return out_pad[:, :NOUT]
